# Initial kernel scaffold; baseline (speedup 1.0000x reference)
import functools

import jax
import jax.numpy as jnp
from jax import lax
from jax.experimental import pallas as pl
from jax.experimental.pallas import tpu as pltpu
from jax.experimental.pallas import tpu_sc as plsc

_BLK = 128
_NW = 32
_CH = 128


def _round_up(a, b):
    return (a + b - 1) // b * b


def _make_propagate(NP, EP, F):
    EPW = EP // _NW
    NCH = EPW // _CH
    RPT = NP // 16
    ZR = 64
    NZ = RPT // ZR
    mesh = plsc.VectorSubcoreMesh(core_axis_name="c", subcore_axis_name="s")

    def body(u_hbm, src_hbm, dst_hbm, out_hbm, sidx, didx, rows, zbuf, acc, sem):
        c = lax.axis_index("c")
        s = lax.axis_index("s")
        wid = s * 2 + c

        def zrow(i, _):
            def zcol(j, _):
                zbuf[i, pl.ds(j * 16, 16)] = jnp.zeros((16,), jnp.float32)
                return 0
            return lax.fori_loop(0, F // 16, zcol, 0)
        lax.fori_loop(0, ZR, zrow, 0)

        def zacc(k, _):
            pltpu.sync_copy(zbuf, acc.at[pl.ds(s * RPT + k * ZR, ZR)])
            return 0
        lax.fori_loop(0, NZ, zacc, 0)
        plsc.subcore_barrier()

        base = wid * EPW

        def chunk(i, _):
            off = base + i * _CH
            pltpu.sync_copy(src_hbm.at[pl.ds(off, _CH)], sidx)
            pltpu.sync_copy(dst_hbm.at[pl.ds(off, _CH)], didx)
            pltpu.async_copy(u_hbm.at[sidx], rows, sem).wait()
            pltpu.sync_copy(rows, acc.at[didx], add=True)
            return 0
        lax.fori_loop(0, NCH, chunk, 0)
        plsc.subcore_barrier()

        pltpu.sync_copy(acc.at[pl.ds(s * RPT, RPT)],
                        out_hbm.at[pl.ds(c * NP + s * RPT, RPT)])

    return pl.kernel(
        body,
        out_type=jax.ShapeDtypeStruct((2 * NP, F), jnp.float32),
        mesh=mesh,
        scratch_types=[
            pltpu.VMEM((_CH,), jnp.int32),
            pltpu.VMEM((_CH,), jnp.int32),
            pltpu.VMEM((_CH, F), jnp.float32),
            pltpu.VMEM((ZR, F), jnp.float32),
            pltpu.VMEM_SHARED((NP, F), jnp.float32),
            pltpu.SemaphoreType.DMA,
        ],
    )


def _make_gather(NP, W):
    mesh = plsc.VectorSubcoreMesh(core_axis_name="c", subcore_axis_name="s")

    def body(x_hbm, idx_hbm, out_hbm, idxv, rows, sem):
        c = lax.axis_index("c")
        s = lax.axis_index("s")

        @pl.when(jnp.logical_and(c == 0, s == 0))
        def _():
            pltpu.sync_copy(idx_hbm, idxv)
            pltpu.async_copy(x_hbm.at[idxv], rows, sem).wait()
            pltpu.sync_copy(rows, out_hbm)

    return pl.kernel(
        body,
        out_type=jax.ShapeDtypeStruct((128, W), jnp.float32),
        mesh=mesh,
        scratch_types=[
            pltpu.VMEM((128,), jnp.int32),
            pltpu.VMEM((128, W), jnp.float32),
            pltpu.SemaphoreType.DMA,
        ],
    )


def _mm_nt(a, b):
    return lax.dot_general(a, b, (((1,), (1,)), ((), ())),
                           preferred_element_type=jnp.float32)


def _diag(v):
    col = jnp.dot(jnp.ones((_BLK, 1), jnp.float32), v,
                  preferred_element_type=jnp.float32)
    i = lax.broadcasted_iota(jnp.int32, (_BLK, _BLK), 0)
    j = lax.broadcasted_iota(jnp.int32, (_BLK, _BLK), 1)
    return jnp.where(i == j, col, 0.0)


def _gelu(x):
    return 0.5 * x * (1.0 + lax.erf(x * 0.7071067811865476))


def _foot_body(N, x_ref, da_ref, db_ref, wf_ref, bf_ref, w0_ref,
               xx_ref, u_ref, dis_ref):
    r = pl.program_id(0)
    degv = da_ref[...] + db_ref[...] + 1.0
    lane = lax.broadcasted_iota(jnp.int32, (1, _BLK), 1)
    disv = jnp.where(r * _BLK + lane < N, lax.rsqrt(degv), 0.0)
    rowi = lax.broadcasted_iota(jnp.int32, (_BLK, 1), 0)
    validr = r * _BLK + rowi < N
    h = _mm_nt(x_ref[...], wf_ref[...]) + bf_ref[...]
    xx = jnp.where(validr, _gelu(h), 0.0)
    xx_ref[...] = xx
    u_ref[...] = jnp.dot(_diag(disv), _mm_nt(xx, w0_ref[...]),
                         preferred_element_type=jnp.float32)
    dis_ref[...] = disv


def _update_body(N, xx_ref, u_ref, sa_ref, sb_ref, dis_ref, b_ref, w_ref,
                 xxo_ref, uo_ref):
    r = pl.program_id(0)
    D = _diag(dis_ref[...])
    ssum = sa_ref[...] + sb_ref[...] + u_ref[...]
    xn = xx_ref[...] + jnp.dot(D, ssum, preferred_element_type=jnp.float32) \
        + b_ref[...]
    rowi = lax.broadcasted_iota(jnp.int32, (_BLK, 1), 0)
    xn = jnp.where(r * _BLK + rowi < N, xn, 0.0)
    xxo_ref[...] = xn
    uo_ref[...] = jnp.dot(D, _mm_nt(xn, w_ref[...]),
                          preferred_element_type=jnp.float32)


def _head_body(rows_ref, wh_ref, bh_ref, o_ref):
    o_ref[...] = _mm_nt(_gelu(rows_ref[...]), wh_ref[...]) + bh_ref[...]


def kernel(x, edge_index, center, ptr, W_foot, b_foot, W_blocks, b_blocks,
           W_head, b_head):
    N, DIN = x.shape
    E = edge_index.shape[1]
    WIDTH = W_foot.shape[0]
    B = center.shape[0]
    NC = W_head.shape[0]
    NP = _round_up(N + 1, 1024)
    EP = _round_up(E, _CH * _NW)
    nblk = NP // _BLK

    pad = jnp.full((EP - E,), N, jnp.int32)
    srcp = jnp.concatenate([edge_index[0].astype(jnp.int32), pad])
    dstp = jnp.concatenate([edge_index[1].astype(jnp.int32), pad])
    xpad = jnp.zeros((NP, DIN), jnp.float32).at[:N].set(x)
    validc = (jnp.arange(NP) < N).astype(jnp.float32)[:, None]
    ones16 = jnp.broadcast_to(validc, (NP, 16))

    deg2 = _make_propagate(NP, EP, 16)(ones16, srcp, dstp)
    dega = deg2[:NP, 0].reshape(nblk, _BLK)
    degb = deg2[NP:, 0].reshape(nblk, _BLK)

    row_spec = pl.BlockSpec((_BLK, WIDTH), lambda r: (r, 0))
    vec_spec = pl.BlockSpec((1, _BLK), lambda r: (r, 0))
    w_spec = pl.BlockSpec((WIDTH, WIDTH), lambda r: (0, 0))
    b_spec = pl.BlockSpec((1, WIDTH), lambda r: (0, 0))

    xx, u, dis = pl.pallas_call(
        functools.partial(_foot_body, N),
        grid=(nblk,),
        in_specs=[pl.BlockSpec((_BLK, DIN), lambda r: (r, 0)),
                  vec_spec, vec_spec,
                  pl.BlockSpec((WIDTH, DIN), lambda r: (0, 0)),
                  b_spec, w_spec],
        out_specs=[row_spec, row_spec, vec_spec],
        out_shape=[jax.ShapeDtypeStruct((NP, WIDTH), jnp.float32),
                   jax.ShapeDtypeStruct((NP, WIDTH), jnp.float32),
                   jax.ShapeDtypeStruct((nblk, _BLK), jnp.float32)],
    )(xpad, dega, degb, W_foot, b_foot.reshape(1, -1), W_blocks[0])

    prop = _make_propagate(NP, EP, WIDTH)
    n_layers = W_blocks.shape[0]
    for l in range(n_layers):
        s2 = prop(u, srcp, dstp)
        wn = W_blocks[l + 1] if l + 1 < n_layers else W_blocks[0]
        xx, u = pl.pallas_call(
            functools.partial(_update_body, N),
            grid=(nblk,),
            in_specs=[row_spec, row_spec,
                      row_spec,
                      pl.BlockSpec((_BLK, WIDTH), lambda r: (r + nblk, 0)),
                      vec_spec, b_spec, w_spec],
            out_specs=[row_spec, row_spec],
            out_shape=[jax.ShapeDtypeStruct((NP, WIDTH), jnp.float32),
                       jax.ShapeDtypeStruct((NP, WIDTH), jnp.float32)],
        )(xx, u, s2, s2, dis, b_blocks[l].reshape(1, -1), wn)

    idxp = jnp.concatenate([(center + ptr[:-1]).astype(jnp.int32),
                            jnp.zeros((128 - B,), jnp.int32)])
    rows = _make_gather(NP, WIDTH)(xx, idxp)

    whp = jnp.zeros((_BLK, WIDTH), jnp.float32).at[:NC].set(W_head)
    bhp = jnp.zeros((1, _BLK), jnp.float32).at[0, :NC].set(b_head)
    out = pl.pallas_call(
        _head_body,
        out_shape=jax.ShapeDtypeStruct((128, _BLK), jnp.float32),
    )(rows, whp, bhp)
    return out[:B, :NC]


# speedup vs baseline: 6.4743x; 6.4743x over previous
import functools

import jax
import jax.numpy as jnp
from jax import lax
from jax.experimental import pallas as pl
from jax.experimental.pallas import tpu as pltpu
from jax.experimental.pallas import tpu_sc as plsc

_BLK = 128
_NW = 32
_CH = 128


def _round_up(a, b):
    return (a + b - 1) // b * b


def _make_propagate(NP, EP, F):
    EPW = EP // _NW
    NCH = EPW // _CH
    RPT = NP // 16
    ZR = 64
    NZ = RPT // ZR
    mesh = plsc.VectorSubcoreMesh(core_axis_name="c", subcore_axis_name="s",
                                  num_cores=2, num_subcores=16)

    def body(u_hbm, src_hbm, dst_hbm, out_hbm, sidx, didx, rows, zbuf, acc, sem):
        c = lax.axis_index("c")
        s = lax.axis_index("s")
        wid = s * 2 + c

        def zrow(i, _):
            def zcol(j, _):
                zbuf[i, pl.ds(j * 16, 16)] = jnp.zeros((16,), jnp.float32)
                return 0
            return lax.fori_loop(0, F // 16, zcol, 0)
        lax.fori_loop(0, ZR, zrow, 0)

        def zacc(k, _):
            pltpu.sync_copy(zbuf, acc.at[pl.ds(s * RPT + k * ZR, ZR)])
            return 0
        lax.fori_loop(0, NZ, zacc, 0)
        plsc.subcore_barrier()

        base = wid * EPW

        def chunk(i, _):
            off = base + i * _CH
            pltpu.sync_copy(src_hbm.at[pl.ds(off, _CH)], sidx)
            pltpu.sync_copy(dst_hbm.at[pl.ds(off, _CH)], didx)
            pltpu.async_copy(u_hbm.at[sidx], rows, sem).wait()
            pltpu.sync_copy(rows, acc.at[didx], add=True)
            return 0
        lax.fori_loop(0, NCH, chunk, 0)
        plsc.subcore_barrier()

        pltpu.sync_copy(acc.at[pl.ds(s * RPT, RPT)],
                        out_hbm.at[pl.ds(c * NP + s * RPT, RPT)])

    return pl.kernel(
        body,
        out_type=jax.ShapeDtypeStruct((2 * NP, F), jnp.float32),
        mesh=mesh,
        compiler_params=pltpu.CompilerParams(
            use_tc_tiling_on_sc=(F % 128 == 0)),
        scratch_types=[
            pltpu.VMEM((_CH,), jnp.int32),
            pltpu.VMEM((_CH,), jnp.int32),
            pltpu.VMEM((_CH, F), jnp.float32),
            pltpu.VMEM((ZR, F), jnp.float32),
            pltpu.VMEM_SHARED((NP, F), jnp.float32),
            pltpu.SemaphoreType.DMA,
        ],
    )


def _make_gather(NP, W):
    mesh = plsc.VectorSubcoreMesh(core_axis_name="c", subcore_axis_name="s",
                                  num_cores=2, num_subcores=16)

    def body(x_hbm, idx_hbm, out_hbm, idxv, rows, sem):
        c = lax.axis_index("c")
        s = lax.axis_index("s")

        @pl.when(jnp.logical_and(c == 0, s == 0))
        def _():
            pltpu.sync_copy(idx_hbm, idxv)
            pltpu.async_copy(x_hbm.at[idxv], rows, sem).wait()
            pltpu.sync_copy(rows, out_hbm)

    return pl.kernel(
        body,
        out_type=jax.ShapeDtypeStruct((128, W), jnp.float32),
        mesh=mesh,
        scratch_types=[
            pltpu.VMEM((128,), jnp.int32),
            pltpu.VMEM((128, W), jnp.float32),
            pltpu.SemaphoreType.DMA,
        ],
    )


def _mm_nt(a, b):
    return lax.dot_general(a, b, (((1,), (1,)), ((), ())),
                           preferred_element_type=jnp.float32)


def _diag(v):
    col = jnp.dot(jnp.ones((_BLK, 1), jnp.float32), v,
                  preferred_element_type=jnp.float32)
    i = lax.broadcasted_iota(jnp.int32, (_BLK, _BLK), 0)
    j = lax.broadcasted_iota(jnp.int32, (_BLK, _BLK), 1)
    return jnp.where(i == j, col, 0.0)


def _gelu(x):
    return 0.5 * x * (1.0 + lax.erf(x * 0.7071067811865476))


def _foot_body(N, x_ref, da_ref, db_ref, wf_ref, bf_ref, w0_ref,
               xx_ref, u_ref, dis_ref):
    r = pl.program_id(0)
    degv = da_ref[0] + db_ref[0] + 1.0
    lane = lax.broadcasted_iota(jnp.int32, (1, _BLK), 1)
    disv = jnp.where(r * _BLK + lane < N, lax.rsqrt(degv), 0.0)
    rowi = lax.broadcasted_iota(jnp.int32, (_BLK, 1), 0)
    validr = r * _BLK + rowi < N
    h = _mm_nt(x_ref[...], wf_ref[...]) + bf_ref[...]
    xx = jnp.where(validr, _gelu(h), 0.0)
    xx_ref[...] = xx
    u_ref[...] = jnp.dot(_diag(disv), _mm_nt(xx, w0_ref[...]),
                         preferred_element_type=jnp.float32)
    dis_ref[0] = disv


def _update_body(N, xx_ref, u_ref, sa_ref, sb_ref, dis_ref, b_ref, w_ref,
                 xxo_ref, uo_ref):
    r = pl.program_id(0)
    D = _diag(dis_ref[0])
    ssum = sa_ref[...] + sb_ref[...] + u_ref[...]
    xn = xx_ref[...] + jnp.dot(D, ssum, preferred_element_type=jnp.float32) \
        + b_ref[...]
    rowi = lax.broadcasted_iota(jnp.int32, (_BLK, 1), 0)
    xn = jnp.where(r * _BLK + rowi < N, xn, 0.0)
    xxo_ref[...] = xn
    uo_ref[...] = jnp.dot(D, _mm_nt(xn, w_ref[...]),
                          preferred_element_type=jnp.float32)


def _head_body(rows_ref, wh_ref, bh_ref, o_ref):
    o_ref[...] = _mm_nt(_gelu(rows_ref[...]), wh_ref[...]) + bh_ref[...]


def kernel(x, edge_index, center, ptr, W_foot, b_foot, W_blocks, b_blocks,
           W_head, b_head):
    N, DIN = x.shape
    E = edge_index.shape[1]
    WIDTH = W_foot.shape[0]
    B = center.shape[0]
    NC = W_head.shape[0]
    NP = _round_up(N + 1, 1024)
    EP = _round_up(E, _CH * _NW)
    nblk = NP // _BLK

    pad = jnp.full((EP - E,), N, jnp.int32)
    srcp = jnp.concatenate([edge_index[0].astype(jnp.int32), pad])
    dstp = jnp.concatenate([edge_index[1].astype(jnp.int32), pad])
    xpad = jnp.zeros((NP, DIN), jnp.float32).at[:N].set(x)
    validc = (jnp.arange(NP) < N).astype(jnp.float32)[:, None]
    ones16 = jnp.broadcast_to(validc, (NP, 16))

    deg2 = _make_propagate(NP, EP, 16)(ones16, srcp, dstp)
    dega = deg2[:NP, 0].reshape(nblk, 1, _BLK)
    degb = deg2[NP:, 0].reshape(nblk, 1, _BLK)

    row_spec = pl.BlockSpec((_BLK, WIDTH), lambda r: (r, 0))
    vec_spec = pl.BlockSpec((1, 1, _BLK), lambda r: (r, 0, 0))
    w_spec = pl.BlockSpec((WIDTH, WIDTH), lambda r: (0, 0))
    b_spec = pl.BlockSpec((1, WIDTH), lambda r: (0, 0))

    xx, u, dis = pl.pallas_call(
        functools.partial(_foot_body, N),
        grid=(nblk,),
        in_specs=[pl.BlockSpec((_BLK, DIN), lambda r: (r, 0)),
                  vec_spec, vec_spec,
                  pl.BlockSpec((WIDTH, DIN), lambda r: (0, 0)),
                  b_spec, w_spec],
        out_specs=[row_spec, row_spec, vec_spec],
        out_shape=[jax.ShapeDtypeStruct((NP, WIDTH), jnp.float32),
                   jax.ShapeDtypeStruct((NP, WIDTH), jnp.float32),
                   jax.ShapeDtypeStruct((nblk, 1, _BLK), jnp.float32)],
    )(xpad, dega, degb, W_foot, b_foot.reshape(1, -1), W_blocks[0])

    prop = _make_propagate(NP, EP, WIDTH)
    n_layers = W_blocks.shape[0]
    for l in range(n_layers):
        s2 = prop(u, srcp, dstp)
        wn = W_blocks[l + 1] if l + 1 < n_layers else W_blocks[0]
        xx, u = pl.pallas_call(
            functools.partial(_update_body, N),
            grid=(nblk,),
            in_specs=[row_spec, row_spec,
                      row_spec,
                      pl.BlockSpec((_BLK, WIDTH), lambda r: (r + nblk, 0)),
                      vec_spec, b_spec, w_spec],
            out_specs=[row_spec, row_spec],
            out_shape=[jax.ShapeDtypeStruct((NP, WIDTH), jnp.float32),
                       jax.ShapeDtypeStruct((NP, WIDTH), jnp.float32)],
        )(xx, u, s2, s2, dis, b_blocks[l].reshape(1, -1), wn)

    idxp = jnp.concatenate([(center + ptr[:-1]).astype(jnp.int32),
                            jnp.zeros((128 - B,), jnp.int32)])
    rows = _make_gather(NP, WIDTH)(xx, idxp)

    whp = jnp.zeros((_BLK, WIDTH), jnp.float32).at[:NC].set(W_head)
    bhp = jnp.zeros((1, _BLK), jnp.float32).at[0, :NC].set(b_head)
    out = pl.pallas_call(
        _head_body,
        out_shape=jax.ShapeDtypeStruct((128, _BLK), jnp.float32),
    )(rows, whp, bhp)
    return out[:B, :NC]
